# Initial kernel scaffold; baseline (speedup 1.0000x reference)
import functools

import jax
import jax.numpy as jnp
from jax import lax
from jax.experimental import pallas as pl
from jax.experimental.pallas import tpu as pltpu
from jax.experimental.pallas import tpu_sc as plsc

B = 16384
D = 128
S = 10
E = 128
NC = 2
NS = 16
NW = NC * NS
NPW = B // NW
CHUNK = 32
NCHUNK = NPW // CHUNK
GROWS = CHUNK * S
GSPLIT = 64
NG = GROWS // GSPLIT

_mesh = plsc.VectorSubcoreMesh(
    core_axis_name="c", subcore_axis_name="s", num_cores=NC, num_subcores=NS
)


def _sc_body(nidx_hbm, nodes_hbm, table_hbm, self_out, nsum_out,
             nidx_v, sidx_v, nrows_v, srows_v, acc_v, gsem):
    wid = lax.axis_index("s") * NC + lax.axis_index("c")

    def chunk_body(k, carry):
        base = wid * NPW + k * CHUNK
        pltpu.sync_copy(nidx_hbm.at[pl.ds(base * S, GROWS)], nidx_v)
        pltpu.sync_copy(nodes_hbm.at[pl.ds(base, CHUNK)], sidx_v)
        copies = [
            pltpu.async_copy(
                table_hbm.at[nidx_v.at[pl.ds(g * GSPLIT, GSPLIT)]],
                nrows_v.at[pl.ds(g * GSPLIT, GSPLIT)],
                gsem,
            )
            for g in range(NG)
        ]
        copies.append(pltpu.async_copy(table_hbm.at[sidx_v], srows_v, gsem))
        for cp in copies:
            cp.wait()

        def node_body(i, c2):
            r0 = i * S
            for c in range(D // 16):
                cs = pl.ds(c * 16, 16)
                accv = nrows_v[r0, cs]
                for j in range(1, S):
                    accv = accv + nrows_v[r0 + j, cs]
            return c2

        lax.fori_loop(0, CHUNK, node_body, 0)
        pltpu.sync_copy(acc_v, nsum_out.at[pl.ds(base, CHUNK)])
        pltpu.sync_copy(srows_v, self_out.at[pl.ds(base, CHUNK)])
        return carry

    lax.fori_loop(0, NCHUNK, chunk_body, 0)


def _fix_node_body():
    pass


_sc_gather = pl.kernel(
    _sc_body,
    out_type=[
        jax.ShapeDtypeStruct((B, D), jnp.float32),
        jax.ShapeDtypeStruct((B, D), jnp.float32),
    ],
    mesh=_mesh,
    scratch_types=[
        pltpu.VMEM((GROWS,), jnp.int32),
        pltpu.VMEM((CHUNK,), jnp.int32),
        pltpu.VMEM((GROWS, D), jnp.float32),
        pltpu.VMEM((CHUNK, D), jnp.float32),
        pltpu.VMEM((CHUNK, D), jnp.float32),
        pltpu.SemaphoreType.DMA,
    ],
)

BT = 2048


def _tc_body(ws_ref, wn_ref, s_ref, n_ref, o_ref):
    dn = (((1,), (1,)), ((), ()))
    a = lax.dot_general(ws_ref[...], s_ref[...], dn,
                        preferred_element_type=jnp.float32)
    b = lax.dot_general(wn_ref[...], n_ref[...], dn,
                        preferred_element_type=jnp.float32)
    o_ref[...] = jnp.maximum(a + 0.1 * b, 0.0)


def _tc_call(ws, wn, self_f, nsum):
    return pl.pallas_call(
        _tc_body,
        grid=(B // BT,),
        in_specs=[
            pl.BlockSpec((E, D), lambda j: (0, 0)),
            pl.BlockSpec((E, D), lambda j: (0, 0)),
            pl.BlockSpec((BT, D), lambda j: (j, 0)),
            pl.BlockSpec((BT, D), lambda j: (j, 0)),
        ],
        out_specs=pl.BlockSpec((E, BT), lambda j: (0, j)),
        out_shape=jax.ShapeDtypeStruct((E, B), jnp.float32),
    )(ws, wn, self_f, nsum)


@jax.jit
def kernel(nodes, neigh_idx, feat_table, weight):
    nidx_flat = neigh_idx.reshape(-1).astype(jnp.int32)
    nodes32 = nodes.astype(jnp.int32)
    self_f, nsum = _sc_gather(nidx_flat, nodes32, feat_table)
    ws = weight[:, :D]
    wn = weight[:, D:]
    return _tc_call(ws, wn, self_f, nsum)


# speedup vs baseline: 2.1095x; 2.1095x over previous
import functools

import jax
import jax.numpy as jnp
from jax import lax
from jax.experimental import pallas as pl
from jax.experimental.pallas import tpu as pltpu
from jax.experimental.pallas import tpu_sc as plsc

B = 16384
D = 128
S = 10
E = 128
NC = 2
NS = 16
NW = NC * NS
NPW = B // NW
CHUNK = 32
NCHUNK = NPW // CHUNK
GROWS = CHUNK * S
GSPLIT = 64
NG = GROWS // GSPLIT

_mesh = plsc.VectorSubcoreMesh(
    core_axis_name="c", subcore_axis_name="s", num_cores=NC, num_subcores=NS
)


def _sc_body(nidx_hbm, nodes_hbm, table_hbm, self_out, nsum_out,
             nidx_v, sidx_v, nrows_v, srows_v, acc_v, gsem):
    wid = lax.axis_index("s") * NC + lax.axis_index("c")

    def chunk_body(k, carry):
        base = wid * NPW + k * CHUNK
        pltpu.sync_copy(nidx_hbm.at[pl.ds(base * S, GROWS)], nidx_v)
        pltpu.sync_copy(nodes_hbm.at[pl.ds(base, CHUNK)], sidx_v)
        copies = [
            pltpu.async_copy(
                table_hbm.at[nidx_v.at[pl.ds(g * GSPLIT, GSPLIT)]],
                nrows_v.at[pl.ds(g * GSPLIT, GSPLIT)],
                gsem,
            )
            for g in range(NG)
        ]
        copies.append(pltpu.async_copy(table_hbm.at[sidx_v], srows_v, gsem))
        for cp in copies:
            cp.wait()

        def node_body(i, c2):
            r0 = i * S
            for c in range(D // 16):
                cs = pl.ds(c * 16, 16)
                accv = nrows_v[r0, cs]
                for j in range(1, S):
                    accv = accv + nrows_v[r0 + j, cs]
                acc_v[i, cs] = accv
            return c2

        lax.fori_loop(0, CHUNK, node_body, 0)
        pltpu.sync_copy(acc_v, nsum_out.at[pl.ds(base, CHUNK)])
        pltpu.sync_copy(srows_v, self_out.at[pl.ds(base, CHUNK)])
        return carry

    lax.fori_loop(0, NCHUNK, chunk_body, 0)


_sc_gather = pl.kernel(
    _sc_body,
    out_type=[
        jax.ShapeDtypeStruct((B, D), jnp.float32),
        jax.ShapeDtypeStruct((B, D), jnp.float32),
    ],
    mesh=_mesh,
    scratch_types=[
        pltpu.VMEM((GROWS,), jnp.int32),
        pltpu.VMEM((CHUNK,), jnp.int32),
        pltpu.VMEM((GROWS, D), jnp.float32),
        pltpu.VMEM((CHUNK, D), jnp.float32),
        pltpu.VMEM((CHUNK, D), jnp.float32),
        pltpu.SemaphoreType.DMA,
    ],
)

BT = 2048


def _tc_body(ws_ref, wn_ref, s_ref, n_ref, o_ref):
    dn = (((1,), (1,)), ((), ()))
    a = lax.dot_general(ws_ref[...], s_ref[...], dn,
                        preferred_element_type=jnp.float32)
    b = lax.dot_general(wn_ref[...], n_ref[...], dn,
                        preferred_element_type=jnp.float32)
    o_ref[...] = jnp.maximum(a + 0.1 * b, 0.0)


def _tc_call(ws, wn, self_f, nsum):
    return pl.pallas_call(
        _tc_body,
        grid=(B // BT,),
        in_specs=[
            pl.BlockSpec((E, D), lambda j: (0, 0)),
            pl.BlockSpec((E, D), lambda j: (0, 0)),
            pl.BlockSpec((BT, D), lambda j: (j, 0)),
            pl.BlockSpec((BT, D), lambda j: (j, 0)),
        ],
        out_specs=pl.BlockSpec((E, BT), lambda j: (0, j)),
        out_shape=jax.ShapeDtypeStruct((E, B), jnp.float32),
    )(ws, wn, self_f, nsum)


@jax.jit
def kernel(nodes, neigh_idx, feat_table, weight):
    nidx_flat = neigh_idx.reshape(-1).astype(jnp.int32)
    nodes32 = nodes.astype(jnp.int32)
    self_f, nsum = _sc_gather(nidx_flat, nodes32, feat_table)
    ws = weight[:, :D]
    wn = weight[:, D:]
    return _tc_call(ws, wn, self_f, nsum)
